# Initial kernel scaffold; baseline (speedup 1.0000x reference)
import jax
import jax.numpy as jnp
from jax.experimental import pallas as pl


def kernel(x, table):
    raise NotImplementedError("write your pallas kernel here")


# speedup vs baseline: 2.0164x; 2.0164x over previous
import functools

import jax
import jax.numpy as jnp
from jax import lax
from jax.experimental import pallas as pl
from jax.experimental.pallas import tpu as pltpu
from jax.experimental.pallas import tpu_sc as plsc

EMBED = 128
NC = 2
NS = 16
NW = NC * NS
G = 128


def _sc_gather(xf, table):
    B = xf.shape[0]
    per_w = B // NW
    groups = per_w // G
    mesh = plsc.VectorSubcoreMesh(core_axis_name="c", subcore_axis_name="s")

    @functools.partial(
        pl.kernel,
        mesh=mesh,
        out_type=jax.ShapeDtypeStruct((B, EMBED), jnp.float32),
        scratch_types=[
            pltpu.VMEM((G,), jnp.int32),
            pltpu.VMEM((G, EMBED), jnp.float32),
            pltpu.SemaphoreType.DMA,
        ],
    )
    def k(idx_hbm, table_hbm, out_hbm, idx_v, rows_v, sem):
        wid = lax.axis_index("s") * NC + lax.axis_index("c")
        base = wid * per_w

        def body(g, carry):
            off = base + g * G
            pltpu.sync_copy(idx_hbm.at[pl.ds(off, G)], idx_v)
            pltpu.async_copy(table_hbm.at[idx_v], rows_v, sem).wait()
            pltpu.sync_copy(rows_v, out_hbm.at[pl.ds(off, G)])
            return carry

        lax.fori_loop(0, groups, body, 0)

    return k(xf, table)


def kernel(x, table):
    n, s = x.shape
    xf = x.reshape(n * s)
    out = _sc_gather(xf, table)
    return out.reshape(n, s, EMBED)


# speedup vs baseline: 2.0224x; 1.0030x over previous
import functools

import jax
import jax.numpy as jnp
from jax import lax
from jax.experimental import pallas as pl
from jax.experimental.pallas import tpu as pltpu
from jax.experimental.pallas import tpu_sc as plsc

EMBED = 128
NC = 2
NS = 16
NW = NC * NS
G = 128
NB = 4
IC = 20


def _sc_gather(xf2, table):
    rows_total, g = xf2.shape
    assert g == G
    B = rows_total * G
    per_w_rows = rows_total // NW
    nb_iter = per_w_rows // NB
    assert per_w_rows % (NB * IC) == 0
    mesh = plsc.VectorSubcoreMesh(core_axis_name="c", subcore_axis_name="s")

    scratch = [
        pltpu.VMEM((IC * NB, G), jnp.int32),
        pltpu.VMEM((NB, G, EMBED), jnp.float32),
    ] + [pltpu.SemaphoreType.DMA] * (2 * NB)

    @functools.partial(
        pl.kernel,
        mesh=mesh,
        out_type=jax.ShapeDtypeStruct((B, EMBED), jnp.float32),
        scratch_types=scratch,
    )
    def k(idx_hbm, table_hbm, out_hbm, idx_v, rows, *sems):
        g_sems = sems[:NB]
        st_sems = sems[NB:]
        wid = lax.axis_index("s") * NC + lax.axis_index("c")
        base_row = wid * per_w_rows

        def body(j, carry):
            row0 = base_row + j * NB

            @pl.when(lax.rem(j, IC) == 0)
            def _stage():
                stage_row = pl.multiple_of(
                    base_row + lax.div(j, IC) * (IC * NB), 8)
                pltpu.sync_copy(idx_hbm.at[pl.ds(stage_row, IC * NB)], idx_v)

            ib = lax.rem(j, IC) * NB
            for b in range(NB):
                @pl.when(j > 0)
                def _drain(b=b):
                    pltpu.make_async_copy(
                        rows.at[b], out_hbm.at[pl.ds((row0 + b) * G, G)],
                        st_sems[b]).wait()
                pltpu.async_copy(table_hbm.at[idx_v.at[ib + b]], rows.at[b],
                                 g_sems[b])
            for b in range(NB):
                pltpu.make_async_copy(table_hbm.at[idx_v.at[ib + b]],
                                      rows.at[b], g_sems[b]).wait()
                pltpu.async_copy(rows.at[b], out_hbm.at[pl.ds((row0 + b) * G, G)],
                                 st_sems[b])
            return carry

        lax.fori_loop(0, nb_iter, body, 0)
        row_last = base_row + (nb_iter - 1) * NB
        for b in range(NB):
            pltpu.make_async_copy(
                rows.at[b], out_hbm.at[pl.ds((row_last + b) * G, G)],
                st_sems[b]).wait()

    return k(xf2, table)


def kernel(x, table):
    n, s = x.shape
    xf2 = x.reshape((n * s) // G, G)
    out = _sc_gather(xf2, table)
    return out.reshape(n, s, EMBED)


# speedup vs baseline: 16.0366x; 7.9293x over previous
import functools

import jax
import jax.numpy as jnp
from jax import lax
from jax.experimental import pallas as pl
from jax.experimental.pallas import tpu as pltpu
from jax.experimental.pallas import tpu_sc as plsc

EMBED = 128
NC = 2
NS = 16
NW = NC * NS
G = 128
NB = 4
IC = 20


def _sc_gather(xf2, table):
    rows_total, g = xf2.shape
    assert g == G
    B = rows_total * G
    per_w_rows = rows_total // NW
    nb_iter = per_w_rows // NB
    assert per_w_rows % (NB * IC) == 0
    mesh = plsc.VectorSubcoreMesh(core_axis_name="c", subcore_axis_name="s")

    scratch = [
        pltpu.VMEM((IC * NB, G), jnp.int32),
        pltpu.VMEM((NB, G, EMBED), jnp.float32),
        pltpu.VMEM_SHARED((32, EMBED), jnp.float32),
    ] + [pltpu.SemaphoreType.DMA] * (2 * NB)

    @functools.partial(
        pl.kernel,
        mesh=mesh,
        out_type=jax.ShapeDtypeStruct((B, EMBED), jnp.float32),
        scratch_types=scratch,
    )
    def k(idx_hbm, table_hbm, out_hbm, idx_v, rows, table_sp, *sems):
        g_sems = sems[:NB]
        st_sems = sems[NB:]
        wid = lax.axis_index("s") * NC + lax.axis_index("c")
        base_row = wid * per_w_rows

        @pl.when(lax.axis_index("s") == 0)
        def _load_table():
            pltpu.sync_copy(table_hbm, table_sp)

        plsc.subcore_barrier()

        def body(j, carry):
            row0 = base_row + j * NB

            @pl.when(lax.rem(j, IC) == 0)
            def _stage():
                stage_row = pl.multiple_of(
                    base_row + lax.div(j, IC) * (IC * NB), 8)
                pltpu.sync_copy(idx_hbm.at[pl.ds(stage_row, IC * NB)], idx_v)

            ib = lax.rem(j, IC) * NB
            for b in range(NB):
                @pl.when(j > 0)
                def _drain(b=b):
                    pltpu.make_async_copy(
                        rows.at[b], out_hbm.at[pl.ds((row0 + b) * G, G)],
                        st_sems[b]).wait()
            for b in range(NB):
                pltpu.async_copy(table_sp.at[idx_v.at[ib + b]], rows.at[b],
                                 g_sems[b])
            for b in range(NB):
                pltpu.make_async_copy(table_sp.at[idx_v.at[ib + b]],
                                      rows.at[b], g_sems[b]).wait()
                pltpu.async_copy(rows.at[b], out_hbm.at[pl.ds((row0 + b) * G, G)],
                                 st_sems[b])
            return carry

        lax.fori_loop(0, nb_iter, body, 0)
        row_last = base_row + (nb_iter - 1) * NB
        for b in range(NB):
            pltpu.make_async_copy(
                rows.at[b], out_hbm.at[pl.ds((row_last + b) * G, G)],
                st_sems[b]).wait()

    return k(xf2, table)


def kernel(x, table):
    n, s = x.shape
    xf2 = x.reshape((n * s) // G, G)
    out = _sc_gather(xf2, table)
    return out.reshape(n, s, EMBED)


# speedup vs baseline: 19.2056x; 1.1976x over previous
import functools

import jax
import jax.numpy as jnp
from jax import lax
from jax.experimental import pallas as pl
from jax.experimental.pallas import tpu as pltpu
from jax.experimental.pallas import tpu_sc as plsc

EMBED = 128
NC = 2
NS = 16
NW = NC * NS
G = 128
NB = 5
IC = 16


def _sc_gather(xf2, table):
    rows_total, g = xf2.shape
    assert g == G
    B = rows_total * G
    per_w_rows = rows_total // NW
    nb_iter = per_w_rows // NB
    nchunk = per_w_rows // (NB * IC)
    assert per_w_rows % (NB * IC) == 0
    mesh = plsc.VectorSubcoreMesh(core_axis_name="c", subcore_axis_name="s")

    scratch = [
        pltpu.VMEM((2, IC * NB, G), jnp.int32),
        pltpu.VMEM((NB, G, EMBED), jnp.float32),
        pltpu.VMEM_SHARED((32, EMBED), jnp.float32),
        pltpu.SemaphoreType.DMA,
    ] + [pltpu.SemaphoreType.DMA] * (2 * NB)

    @functools.partial(
        pl.kernel,
        mesh=mesh,
        out_type=jax.ShapeDtypeStruct((B, EMBED), jnp.float32),
        scratch_types=scratch,
    )
    def k(idx_hbm, table_hbm, out_hbm, idx_v, rows, table_sp, i_sem, *sems):
        g_sems = sems[:NB]
        st_sems = sems[NB:]
        wid = lax.axis_index("s") * NC + lax.axis_index("c")
        base_row = wid * per_w_rows
        CH = IC * NB

        @pl.when(lax.axis_index("s") == 0)
        def _load_table():
            pltpu.sync_copy(table_hbm, table_sp)

        pltpu.async_copy(idx_hbm.at[pl.ds(base_row, CH)], idx_v.at[0], i_sem)
        plsc.subcore_barrier()

        def body(j, carry):
            row0 = base_row + j * NB
            t = lax.div(j, IC)
            par = lax.rem(t, 2)

            @pl.when(lax.rem(j, IC) == 0)
            def _stage():
                pltpu.make_async_copy(
                    idx_hbm.at[pl.ds(pl.multiple_of(base_row + t * CH, 8), CH)],
                    idx_v.at[par], i_sem).wait()

                @pl.when(t + 1 < nchunk)
                def _prefetch():
                    pltpu.async_copy(
                        idx_hbm.at[pl.ds(
                            pl.multiple_of(base_row + (t + 1) * CH, 8), CH)],
                        idx_v.at[1 - par], i_sem)

            ib = lax.rem(j, IC) * NB
            for b in range(NB):
                @pl.when(j > 0)
                def _drain(b=b):
                    pltpu.make_async_copy(
                        rows.at[b], out_hbm.at[pl.ds((row0 + b) * G, G)],
                        st_sems[b]).wait()
                pltpu.async_copy(table_sp.at[idx_v.at[par, ib + b]], rows.at[b],
                                 g_sems[b])
            for b in range(NB):
                pltpu.make_async_copy(table_sp.at[idx_v.at[par, ib + b]],
                                      rows.at[b], g_sems[b]).wait()
                pltpu.async_copy(rows.at[b], out_hbm.at[pl.ds((row0 + b) * G, G)],
                                 st_sems[b])
            return carry

        lax.fori_loop(0, nb_iter, body, 0)
        row_last = base_row + (nb_iter - 1) * NB
        for b in range(NB):
            pltpu.make_async_copy(
                rows.at[b], out_hbm.at[pl.ds((row_last + b) * G, G)],
                st_sems[b]).wait()

    return k(xf2, table)


def kernel(x, table):
    n, s = x.shape
    xf2 = x.reshape((n * s) // G, G)
    out = _sc_gather(xf2, table)
    return out.reshape(n, s, EMBED)


# speedup vs baseline: 19.4428x; 1.0124x over previous
import functools

import jax
import jax.numpy as jnp
from jax import lax
from jax.experimental import pallas as pl
from jax.experimental.pallas import tpu as pltpu
from jax.experimental.pallas import tpu_sc as plsc

EMBED = 128
NC = 2
NS = 16
NW = NC * NS
G = 64
NB = 10
IC = 8


def _sc_gather(xf2, table):
    rows_total, g = xf2.shape
    assert g == G
    B = rows_total * G
    per_w_rows = rows_total // NW
    nb_iter = per_w_rows // NB
    nchunk = per_w_rows // (NB * IC)
    assert per_w_rows % (NB * IC) == 0
    mesh = plsc.VectorSubcoreMesh(core_axis_name="c", subcore_axis_name="s")

    scratch = [
        pltpu.VMEM((2, IC * NB, G), jnp.int32),
        pltpu.VMEM((NB, G, EMBED), jnp.float32),
        pltpu.VMEM_SHARED((32, EMBED), jnp.float32),
        pltpu.SemaphoreType.DMA,
    ] + [pltpu.SemaphoreType.DMA] * (2 * NB)

    @functools.partial(
        pl.kernel,
        mesh=mesh,
        out_type=jax.ShapeDtypeStruct((B, EMBED), jnp.float32),
        scratch_types=scratch,
    )
    def k(idx_hbm, table_hbm, out_hbm, idx_v, rows, table_sp, i_sem, *sems):
        g_sems = sems[:NB]
        st_sems = sems[NB:]
        wid = lax.axis_index("s") * NC + lax.axis_index("c")
        base_row = wid * per_w_rows
        CH = IC * NB

        @pl.when(lax.axis_index("s") == 0)
        def _load_table():
            pltpu.sync_copy(table_hbm, table_sp)

        pltpu.async_copy(idx_hbm.at[pl.ds(base_row, CH)], idx_v.at[0], i_sem)
        plsc.subcore_barrier()

        def body(j, carry):
            row0 = base_row + j * NB
            t = lax.div(j, IC)
            par = lax.rem(t, 2)

            @pl.when(lax.rem(j, IC) == 0)
            def _stage():
                pltpu.make_async_copy(
                    idx_hbm.at[pl.ds(pl.multiple_of(base_row + t * CH, 8), CH)],
                    idx_v.at[par], i_sem).wait()

                @pl.when(t + 1 < nchunk)
                def _prefetch():
                    pltpu.async_copy(
                        idx_hbm.at[pl.ds(
                            pl.multiple_of(base_row + (t + 1) * CH, 8), CH)],
                        idx_v.at[1 - par], i_sem)

            ib = lax.rem(j, IC) * NB
            for b in range(NB):
                @pl.when(j > 0)
                def _drain(b=b):
                    pltpu.make_async_copy(
                        rows.at[b], out_hbm.at[pl.ds((row0 + b) * G, G)],
                        st_sems[b]).wait()
                pltpu.async_copy(table_sp.at[idx_v.at[par, ib + b]], rows.at[b],
                                 g_sems[b])
            for b in range(NB):
                pltpu.make_async_copy(table_sp.at[idx_v.at[par, ib + b]],
                                      rows.at[b], g_sems[b]).wait()
                pltpu.async_copy(rows.at[b], out_hbm.at[pl.ds((row0 + b) * G, G)],
                                 st_sems[b])
            return carry

        lax.fori_loop(0, nb_iter, body, 0)
        row_last = base_row + (nb_iter - 1) * NB
        for b in range(NB):
            pltpu.make_async_copy(
                rows.at[b], out_hbm.at[pl.ds((row_last + b) * G, G)],
                st_sems[b]).wait()

    return k(xf2, table)


def kernel(x, table):
    n, s = x.shape
    xf2 = x.reshape((n * s) // G, G)
    out = _sc_gather(xf2, table)
    return out.reshape(n, s, EMBED)
